# baseline (speedup 1.0000x reference)
import functools

import jax
import jax.numpy as jnp
from jax import lax
from jax.experimental import pallas as pl
from jax.experimental.pallas import tpu as pltpu
from jax.experimental.pallas import tpu_sc as plsc

N = 10000
E = 320000
D = 128
NC = 2
NS = 16
NW = NC * NS
K = 128
CH = -(-E // (NW * K))
CH0 = 97
CH1 = 2 * CH - CH0
CHM = max(CH0, CH1)
EP = NS * (CH0 + CH1) * K
NPAD = 10240
ZR = 32
BR = 2000


def _agg_body(src0_hbm, dst0_hbm, src1_hbm, dst1_hbm, x_hbm, out_hbm, src_v,
              dst_v, rows_v, zero_v, agg_sh, sem):
    cid = lax.axis_index("c")
    sid = lax.axis_index("s")

    zv = jnp.zeros((16,), jnp.float32)

    def zbody(i, c):
        zero_v[i // 8, pl.ds((i % 8) * 16, 16)] = zv
        return c

    lax.fori_loop(0, ZR * 8, zbody, 0)
    rows_per_tile = NPAD // NS
    for t in range(rows_per_tile // ZR):
        pltpu.sync_copy(zero_v,
                        agg_sh.at[pl.ds(sid * rows_per_tile + t * ZR, ZR)])
    plsc.subcore_barrier()

    def body(j, c):
        pltpu.async_copy(x_hbm.at[src_v.at[j]], rows_v, sem).wait()
        pltpu.sync_copy(rows_v, agg_sh.at[dst_v.at[j]], add=True)
        return c

    @pl.when(cid == 0)
    def _():
        pltpu.sync_copy(src0_hbm.at[sid], src_v.at[pl.ds(0, CH0)])
        pltpu.sync_copy(dst0_hbm.at[sid], dst_v.at[pl.ds(0, CH0)])
        lax.fori_loop(0, CH0, body, 0)

    @pl.when(cid == 1)
    def _():
        pltpu.sync_copy(src1_hbm.at[sid], src_v.at[pl.ds(0, CH1)])
        pltpu.sync_copy(dst1_hbm.at[sid], dst_v.at[pl.ds(0, CH1)])
        lax.fori_loop(0, CH1, body, 0)

    plsc.subcore_barrier()
    ro = NPAD // NS
    pltpu.sync_copy(agg_sh.at[pl.ds(sid * ro, ro)],
                    out_hbm.at[cid, pl.ds(sid * ro, ro)])


@functools.partial(
    pl.kernel,
    out_type=jax.ShapeDtypeStruct((NC, NPAD, D), jnp.float32),
    mesh=plsc.VectorSubcoreMesh(core_axis_name="c", subcore_axis_name="s"),
    scratch_types=[
        pltpu.VMEM((CHM, K), jnp.int32),
        pltpu.VMEM((CHM, K), jnp.int32),
        pltpu.VMEM((K, D), jnp.float32),
        pltpu.VMEM((ZR, D), jnp.float32),
        pltpu.VMEM_SHARED((NPAD, D), jnp.float32),
        pltpu.SemaphoreType.DMA,
    ],
)
def _agg(src0_hbm, dst0_hbm, src1_hbm, dst1_hbm, x_hbm, out_hbm, src_v,
         dst_v, rows_v, zero_v, agg_sh, sem):
    _agg_body(src0_hbm, dst0_hbm, src1_hbm, dst1_hbm, x_hbm, out_hbm, src_v,
              dst_v, rows_v, zero_v, agg_sh, sem)


def _mlp_body(scale_ref, h_ref, p0_ref, p1_ref, Wa_ref, ba_ref, Wb_ref,
              bb_ref, o_ref):
    z = scale_ref[0, 0] * h_ref[...] + (p0_ref[...] + p1_ref[...])
    t = jnp.dot(z, Wa_ref[...], preferred_element_type=jnp.float32)
    t = jnp.maximum(t + ba_ref[...], 0.0)
    o_ref[...] = jnp.dot(t, Wb_ref[...],
                         preferred_element_type=jnp.float32) + bb_ref[...]


def _mlp(scale, h, p0, p1, Wa, ba, Wb, bb):
    return pl.pallas_call(
        _mlp_body,
        grid=(N // BR,),
        in_specs=[
            pl.BlockSpec(memory_space=pltpu.SMEM),
            pl.BlockSpec((BR, D), lambda i: (i, 0)),
            pl.BlockSpec((BR, D), lambda i: (i, 0)),
            pl.BlockSpec((BR, D), lambda i: (i, 0)),
            pl.BlockSpec((D, D), lambda i: (0, 0)),
            pl.BlockSpec((1, D), lambda i: (0, 0)),
            pl.BlockSpec((D, D), lambda i: (0, 0)),
            pl.BlockSpec((1, D), lambda i: (0, 0)),
        ],
        out_specs=pl.BlockSpec((BR, D), lambda i: (i, 0)),
        out_shape=jax.ShapeDtypeStruct((N, D), jnp.float32),
    )(scale, h, p0, p1, Wa, ba, Wb, bb)


def _final_body(x_ref, h1_ref, h2_ref, h3_ref, Wf_ref, bf_ref, o_ref):
    acc = jnp.dot(x_ref[...], Wf_ref[0], preferred_element_type=jnp.float32)
    acc += jnp.dot(h1_ref[...], Wf_ref[1], preferred_element_type=jnp.float32)
    acc += jnp.dot(h2_ref[...], Wf_ref[2], preferred_element_type=jnp.float32)
    acc += jnp.dot(h3_ref[...], Wf_ref[3], preferred_element_type=jnp.float32)
    o_ref[...] = acc + bf_ref[...]


def _final(x, h1, h2, h3, Wf4, bf):
    return pl.pallas_call(
        _final_body,
        grid=(N // BR,),
        in_specs=[
            pl.BlockSpec((BR, D), lambda i: (i, 0)),
            pl.BlockSpec((BR, D), lambda i: (i, 0)),
            pl.BlockSpec((BR, D), lambda i: (i, 0)),
            pl.BlockSpec((BR, D), lambda i: (i, 0)),
            pl.BlockSpec((4, D, D), lambda i: (0, 0, 0)),
            pl.BlockSpec((1, D), lambda i: (0, 0)),
        ],
        out_specs=pl.BlockSpec((BR, D), lambda i: (i, 0)),
        out_shape=jax.ShapeDtypeStruct((N, D), jnp.float32),
    )(x, h1, h2, h3, Wf4, bf)


def kernel(x, edge_index, eps1, W1a, b1a, W1b, b1b, eps2, W2a, b2a, W2b, b2b,
           eps3, W3a, b3a, W3b, b3b, Wf, bf):
    src = edge_index[0].astype(jnp.int32)
    dst = edge_index[1].astype(jnp.int32)
    pad = EP - E
    srcp = jnp.concatenate([src, jnp.zeros((pad,), jnp.int32)])
    dstp = jnp.concatenate([dst, jnp.full((pad,), N, jnp.int32)])
    e0 = NS * CH0 * K
    src0 = srcp[:e0].reshape(NS, CH0, K)
    dst0 = dstp[:e0].reshape(NS, CH0, K)
    src1 = srcp[e0:].reshape(NS, CH1, K)
    dst1 = dstp[e0:].reshape(NS, CH1, K)

    def layer(h, eps, Wa, ba, Wb, bb):
        p = _agg(src0, dst0, src1, dst1, h)
        scale = jnp.reshape(1.0 + eps, (1, 1)).astype(jnp.float32)
        return _mlp(scale, h, p[0, :N], p[1, :N], Wa, jnp.reshape(ba, (1, D)),
                    Wb, jnp.reshape(bb, (1, D)))

    h1 = layer(x, eps1, W1a, b1a, W1b, b1b)
    h2 = layer(h1, eps2, W2a, b2a, W2b, b2b)
    h3 = layer(h2, eps3, W3a, b3a, W3b, b3b)
    return _final(x, h1, h2, h3, Wf.reshape(4, D, D), jnp.reshape(bf, (1, D)))


# speedup vs baseline: 1.1245x; 1.1245x over previous
import functools

import jax
import jax.numpy as jnp
from jax import lax
from jax.experimental import pallas as pl
from jax.experimental.pallas import tpu as pltpu
from jax.experimental.pallas import tpu_sc as plsc

N = 10000
E = 320000
D = 128
NC = 2
NS = 16
NW = NC * NS
K = 128
CH = -(-E // (NW * K))
CH0 = 110
CH1 = 2 * CH - CH0
CHM = max(CH0, CH1)
EP = NS * (CH0 + CH1) * K
NPAD = 10240
ZR = 32
BR = 2000


def _agg_body(src0_hbm, dst0_hbm, src1_hbm, dst1_hbm, x_hbm, out_hbm, src_v,
              dst_v, rows_v, zero_v, agg_sh, sem):
    cid = lax.axis_index("c")
    sid = lax.axis_index("s")

    zv = jnp.zeros((16,), jnp.float32)

    def zbody(i, c):
        zero_v[i // 8, pl.ds((i % 8) * 16, 16)] = zv
        return c

    lax.fori_loop(0, ZR * 8, zbody, 0)
    rows_per_tile = NPAD // NS
    for t in range(rows_per_tile // ZR):
        pltpu.sync_copy(zero_v,
                        agg_sh.at[pl.ds(sid * rows_per_tile + t * ZR, ZR)])
    plsc.subcore_barrier()

    def body(j, c):
        pltpu.async_copy(x_hbm.at[src_v.at[j]], rows_v, sem).wait()
        pltpu.sync_copy(rows_v, agg_sh.at[dst_v.at[j]], add=True)
        return c

    @pl.when(cid == 0)
    def _():
        pltpu.sync_copy(src0_hbm.at[sid], src_v.at[pl.ds(0, CH0)])
        pltpu.sync_copy(dst0_hbm.at[sid], dst_v.at[pl.ds(0, CH0)])
        lax.fori_loop(0, CH0, body, 0)

    @pl.when(cid == 1)
    def _():
        pltpu.sync_copy(src1_hbm.at[sid], src_v.at[pl.ds(0, CH1)])
        pltpu.sync_copy(dst1_hbm.at[sid], dst_v.at[pl.ds(0, CH1)])
        lax.fori_loop(0, CH1, body, 0)

    plsc.subcore_barrier()
    ro = NPAD // NS
    pltpu.sync_copy(agg_sh.at[pl.ds(sid * ro, ro)],
                    out_hbm.at[cid, pl.ds(sid * ro, ro)])


@functools.partial(
    pl.kernel,
    out_type=jax.ShapeDtypeStruct((NC, NPAD, D), jnp.float32),
    mesh=plsc.VectorSubcoreMesh(core_axis_name="c", subcore_axis_name="s"),
    scratch_types=[
        pltpu.VMEM((CHM, K), jnp.int32),
        pltpu.VMEM((CHM, K), jnp.int32),
        pltpu.VMEM((K, D), jnp.float32),
        pltpu.VMEM((ZR, D), jnp.float32),
        pltpu.VMEM_SHARED((NPAD, D), jnp.float32),
        pltpu.SemaphoreType.DMA,
    ],
)
def _agg(src0_hbm, dst0_hbm, src1_hbm, dst1_hbm, x_hbm, out_hbm, src_v,
         dst_v, rows_v, zero_v, agg_sh, sem):
    _agg_body(src0_hbm, dst0_hbm, src1_hbm, dst1_hbm, x_hbm, out_hbm, src_v,
              dst_v, rows_v, zero_v, agg_sh, sem)


def _mlp_body(scale_ref, h_ref, p0_ref, p1_ref, Wa_ref, ba_ref, Wb_ref,
              bb_ref, o_ref):
    z = scale_ref[0, 0] * h_ref[...] + (p0_ref[...] + p1_ref[...])
    t = jnp.dot(z, Wa_ref[...], preferred_element_type=jnp.float32)
    t = jnp.maximum(t + ba_ref[...], 0.0)
    o_ref[...] = jnp.dot(t, Wb_ref[...],
                         preferred_element_type=jnp.float32) + bb_ref[...]


def _mlp(scale, h, p0, p1, Wa, ba, Wb, bb):
    return pl.pallas_call(
        _mlp_body,
        grid=(N // BR,),
        in_specs=[
            pl.BlockSpec(memory_space=pltpu.SMEM),
            pl.BlockSpec((BR, D), lambda i: (i, 0)),
            pl.BlockSpec((BR, D), lambda i: (i, 0)),
            pl.BlockSpec((BR, D), lambda i: (i, 0)),
            pl.BlockSpec((D, D), lambda i: (0, 0)),
            pl.BlockSpec((1, D), lambda i: (0, 0)),
            pl.BlockSpec((D, D), lambda i: (0, 0)),
            pl.BlockSpec((1, D), lambda i: (0, 0)),
        ],
        out_specs=pl.BlockSpec((BR, D), lambda i: (i, 0)),
        out_shape=jax.ShapeDtypeStruct((N, D), jnp.float32),
    )(scale, h, p0, p1, Wa, ba, Wb, bb)


def _final_body(x_ref, h1_ref, h2_ref, h3_ref, Wf_ref, bf_ref, o_ref):
    acc = jnp.dot(x_ref[...], Wf_ref[0], preferred_element_type=jnp.float32)
    acc += jnp.dot(h1_ref[...], Wf_ref[1], preferred_element_type=jnp.float32)
    acc += jnp.dot(h2_ref[...], Wf_ref[2], preferred_element_type=jnp.float32)
    acc += jnp.dot(h3_ref[...], Wf_ref[3], preferred_element_type=jnp.float32)
    o_ref[...] = acc + bf_ref[...]


def _final(x, h1, h2, h3, Wf4, bf):
    return pl.pallas_call(
        _final_body,
        grid=(N // BR,),
        in_specs=[
            pl.BlockSpec((BR, D), lambda i: (i, 0)),
            pl.BlockSpec((BR, D), lambda i: (i, 0)),
            pl.BlockSpec((BR, D), lambda i: (i, 0)),
            pl.BlockSpec((BR, D), lambda i: (i, 0)),
            pl.BlockSpec((4, D, D), lambda i: (0, 0, 0)),
            pl.BlockSpec((1, D), lambda i: (0, 0)),
        ],
        out_specs=pl.BlockSpec((BR, D), lambda i: (i, 0)),
        out_shape=jax.ShapeDtypeStruct((N, D), jnp.float32),
    )(x, h1, h2, h3, Wf4, bf)


def kernel(x, edge_index, eps1, W1a, b1a, W1b, b1b, eps2, W2a, b2a, W2b, b2b,
           eps3, W3a, b3a, W3b, b3b, Wf, bf):
    src = edge_index[0].astype(jnp.int32)
    dst = edge_index[1].astype(jnp.int32)
    pad = EP - E
    srcp = jnp.concatenate([src, jnp.zeros((pad,), jnp.int32)])
    dstp = jnp.concatenate([dst, jnp.full((pad,), N, jnp.int32)])
    e0 = NS * CH0 * K
    src0 = srcp[:e0].reshape(NS, CH0, K)
    dst0 = dstp[:e0].reshape(NS, CH0, K)
    src1 = srcp[e0:].reshape(NS, CH1, K)
    dst1 = dstp[e0:].reshape(NS, CH1, K)

    def layer(h, eps, Wa, ba, Wb, bb):
        p = _agg(src0, dst0, src1, dst1, h)
        scale = jnp.reshape(1.0 + eps, (1, 1)).astype(jnp.float32)
        return _mlp(scale, h, p[0, :N], p[1, :N], Wa, jnp.reshape(ba, (1, D)),
                    Wb, jnp.reshape(bb, (1, D)))

    h1 = layer(x, eps1, W1a, b1a, W1b, b1b)
    h2 = layer(h1, eps2, W2a, b2a, W2b, b2b)
    h3 = layer(h2, eps3, W3a, b3a, W3b, b3b)
    return _final(x, h1, h2, h3, Wf.reshape(4, D, D), jnp.reshape(bf, (1, D)))


# speedup vs baseline: 1.1488x; 1.0216x over previous
import functools

import jax
import jax.numpy as jnp
from jax import lax
from jax.experimental import pallas as pl
from jax.experimental.pallas import tpu as pltpu
from jax.experimental.pallas import tpu_sc as plsc

N = 10000
E = 320000
D = 128
NC = 2
NS = 16
NW = NC * NS
K = 128
CH = -(-E // (NW * K))
CH0 = 112
CH1 = 2 * CH - CH0
CHM = max(CH0, CH1)
EP = NS * (CH0 + CH1) * K
NPAD = 10240
ZR = 32
BR = 2000


def _agg_body(src0_hbm, dst0_hbm, src1_hbm, dst1_hbm, x_hbm, out_hbm, src_v,
              dst_v, rows_v, zero_v, agg_sh, sem):
    cid = lax.axis_index("c")
    sid = lax.axis_index("s")

    zv = jnp.zeros((16,), jnp.float32)

    def zbody(i, c):
        zero_v[i // 8, pl.ds((i % 8) * 16, 16)] = zv
        return c

    lax.fori_loop(0, ZR * 8, zbody, 0)
    rows_per_tile = NPAD // NS
    for t in range(rows_per_tile // ZR):
        pltpu.sync_copy(zero_v,
                        agg_sh.at[pl.ds(sid * rows_per_tile + t * ZR, ZR)])
    plsc.subcore_barrier()

    def body(j, c):
        pltpu.async_copy(x_hbm.at[src_v.at[j]], rows_v, sem).wait()
        pltpu.sync_copy(rows_v, agg_sh.at[dst_v.at[j]], add=True)
        return c

    @pl.when(cid == 0)
    def _():
        pltpu.sync_copy(src0_hbm.at[sid], src_v.at[pl.ds(0, CH0)])
        pltpu.sync_copy(dst0_hbm.at[sid], dst_v.at[pl.ds(0, CH0)])
        lax.fori_loop(0, CH0, body, 0)

    @pl.when(cid == 1)
    def _():
        pltpu.sync_copy(src1_hbm.at[sid], src_v.at[pl.ds(0, CH1)])
        pltpu.sync_copy(dst1_hbm.at[sid], dst_v.at[pl.ds(0, CH1)])
        lax.fori_loop(0, CH1, body, 0)

    plsc.subcore_barrier()
    ro = NPAD // NS
    pltpu.sync_copy(agg_sh.at[pl.ds(sid * ro, ro)],
                    out_hbm.at[cid, pl.ds(sid * ro, ro)])


@functools.partial(
    pl.kernel,
    out_type=jax.ShapeDtypeStruct((NC, NPAD, D), jnp.float32),
    mesh=plsc.VectorSubcoreMesh(core_axis_name="c", subcore_axis_name="s"),
    scratch_types=[
        pltpu.VMEM((CHM, K), jnp.int32),
        pltpu.VMEM((CHM, K), jnp.int32),
        pltpu.VMEM((K, D), jnp.float32),
        pltpu.VMEM((ZR, D), jnp.float32),
        pltpu.VMEM_SHARED((NPAD, D), jnp.float32),
        pltpu.SemaphoreType.DMA,
    ],
)
def _agg(src0_hbm, dst0_hbm, src1_hbm, dst1_hbm, x_hbm, out_hbm, src_v,
         dst_v, rows_v, zero_v, agg_sh, sem):
    _agg_body(src0_hbm, dst0_hbm, src1_hbm, dst1_hbm, x_hbm, out_hbm, src_v,
              dst_v, rows_v, zero_v, agg_sh, sem)


def _mlp_body(scale_ref, h_ref, p0_ref, p1_ref, Wa_ref, ba_ref, Wb_ref,
              bb_ref, o_ref):
    z = scale_ref[0, 0] * h_ref[...] + (p0_ref[...] + p1_ref[...])
    t = jnp.dot(z, Wa_ref[...], preferred_element_type=jnp.float32)
    t = jnp.maximum(t + ba_ref[...], 0.0)
    o_ref[...] = jnp.dot(t, Wb_ref[...],
                         preferred_element_type=jnp.float32) + bb_ref[...]


def _mlp(scale, h, p0, p1, Wa, ba, Wb, bb):
    return pl.pallas_call(
        _mlp_body,
        grid=(N // BR,),
        in_specs=[
            pl.BlockSpec(memory_space=pltpu.SMEM),
            pl.BlockSpec((BR, D), lambda i: (i, 0)),
            pl.BlockSpec((BR, D), lambda i: (i, 0)),
            pl.BlockSpec((BR, D), lambda i: (i, 0)),
            pl.BlockSpec((D, D), lambda i: (0, 0)),
            pl.BlockSpec((1, D), lambda i: (0, 0)),
            pl.BlockSpec((D, D), lambda i: (0, 0)),
            pl.BlockSpec((1, D), lambda i: (0, 0)),
        ],
        out_specs=pl.BlockSpec((BR, D), lambda i: (i, 0)),
        out_shape=jax.ShapeDtypeStruct((N, D), jnp.float32),
    )(scale, h, p0, p1, Wa, ba, Wb, bb)


def _final_body(x_ref, h1_ref, h2_ref, h3_ref, Wf_ref, bf_ref, o_ref):
    acc = jnp.dot(x_ref[...], Wf_ref[0], preferred_element_type=jnp.float32)
    acc += jnp.dot(h1_ref[...], Wf_ref[1], preferred_element_type=jnp.float32)
    acc += jnp.dot(h2_ref[...], Wf_ref[2], preferred_element_type=jnp.float32)
    acc += jnp.dot(h3_ref[...], Wf_ref[3], preferred_element_type=jnp.float32)
    o_ref[...] = acc + bf_ref[...]


def _final(x, h1, h2, h3, Wf4, bf):
    return pl.pallas_call(
        _final_body,
        grid=(N // BR,),
        in_specs=[
            pl.BlockSpec((BR, D), lambda i: (i, 0)),
            pl.BlockSpec((BR, D), lambda i: (i, 0)),
            pl.BlockSpec((BR, D), lambda i: (i, 0)),
            pl.BlockSpec((BR, D), lambda i: (i, 0)),
            pl.BlockSpec((4, D, D), lambda i: (0, 0, 0)),
            pl.BlockSpec((1, D), lambda i: (0, 0)),
        ],
        out_specs=pl.BlockSpec((BR, D), lambda i: (i, 0)),
        out_shape=jax.ShapeDtypeStruct((N, D), jnp.float32),
    )(x, h1, h2, h3, Wf4, bf)


def kernel(x, edge_index, eps1, W1a, b1a, W1b, b1b, eps2, W2a, b2a, W2b, b2b,
           eps3, W3a, b3a, W3b, b3b, Wf, bf):
    src = edge_index[0].astype(jnp.int32)
    dst = edge_index[1].astype(jnp.int32)
    pad = EP - E
    srcp = jnp.concatenate([src, jnp.zeros((pad,), jnp.int32)])
    dstp = jnp.concatenate([dst, jnp.full((pad,), N, jnp.int32)])
    e0 = NS * CH0 * K
    src0 = srcp[:e0].reshape(NS, CH0, K)
    dst0 = dstp[:e0].reshape(NS, CH0, K)
    src1 = srcp[e0:].reshape(NS, CH1, K)
    dst1 = dstp[e0:].reshape(NS, CH1, K)

    def layer(h, eps, Wa, ba, Wb, bb):
        p = _agg(src0, dst0, src1, dst1, h)
        scale = jnp.reshape(1.0 + eps, (1, 1)).astype(jnp.float32)
        return _mlp(scale, h, p[0, :N], p[1, :N], Wa, jnp.reshape(ba, (1, D)),
                    Wb, jnp.reshape(bb, (1, D)))

    h1 = layer(x, eps1, W1a, b1a, W1b, b1b)
    h2 = layer(h1, eps2, W2a, b2a, W2b, b2b)
    h3 = layer(h2, eps3, W3a, b3a, W3b, b3b)
    return _final(x, h1, h2, h3, Wf.reshape(4, D, D), jnp.reshape(bf, (1, D)))
